# baseline (speedup 1.0000x reference)
import functools

import jax
import jax.numpy as jnp
from jax import lax
from jax.experimental import pallas as pl
from jax.experimental.pallas import tpu as pltpu
from jax.experimental.pallas import tpu_sc as plsc

_W = 162
_MW = 1088
_NSUB = 32
_TI = 4
_T = _TI * 128


def _ceil_to(n, m):
    return (n + m - 1) // m * m


def _build_lists(wid, n, b):
    indices = jnp.argsort(wid)
    sorted_wid = wid[indices]
    wr = jnp.arange(_W, dtype=jnp.int32)
    starts = jnp.searchsorted(sorted_wid, wr, side="left").astype(jnp.int32)
    ends = jnp.searchsorted(sorted_wid, wr, side="right").astype(jnp.int32)
    counts = ends - starts
    pos = jnp.arange(n, dtype=jnp.int32) - starts[sorted_wid]
    dst_sorted = sorted_wid * _MW + pos

    boff = (jnp.arange(b, dtype=jnp.int32) * (_W * _MW))[:, None]
    src = (jnp.arange(b, dtype=jnp.int32)[:, None] * n
           + indices[None, :].astype(jnp.int32)).reshape(-1)
    dst = (boff + dst_sorted[None, :]).reshape(-1)

    pad_per_w = (_MW - counts).astype(jnp.int32)
    cum_pad = jnp.cumsum(pad_per_w).astype(jnp.int32)
    p0 = _W * _MW - n
    s = jnp.arange(p0, dtype=jnp.int32)
    w_pad = jnp.searchsorted(cum_pad, s, side="right").astype(jnp.int32)
    cum_excl = cum_pad - pad_per_w
    pad0 = w_pad * _MW + counts[w_pad] + (s - cum_excl[w_pad])
    pad = (boff + pad0[None, :]).reshape(-1)
    return indices, counts, src, dst, pad


def _pad_chunk(a):
    m = a.shape[0]
    mp = _ceil_to(m, _T)
    a = jnp.concatenate([a, jnp.full((mp - m,), a[-1], dtype=a.dtype)])
    return a.reshape(mp // 128, 128), mp // _T


def _sc_route(x_flat, src2, dst2, pad2, zeros_rows, n_data_chunks,
              n_pad_chunks, out_rows):
    mesh = plsc.VectorSubcoreMesh(core_axis_name="c", subcore_axis_name="s")

    @functools.partial(
        pl.kernel,
        mesh=mesh,
        out_type=jax.ShapeDtypeStruct((out_rows, 128), jnp.float32),
        scratch_types=[
            pltpu.VMEM((_TI, 128), jnp.int32),
            pltpu.VMEM((_TI, 128), jnp.int32),
            pltpu.VMEM((_T, 128), jnp.float32),
            pltpu.VMEM((128, 128), jnp.float32),
            pltpu.SemaphoreType.DMA,
            pltpu.SemaphoreType.DMA,
        ],
    )
    def k(x_hbm, src_hbm, dst_hbm, pad_hbm, zero_hbm, out_hbm,
          sidx_v, didx_v, data_v, zero_v, sem_g, sem_s):
        sub = lax.axis_index("s") * 2 + lax.axis_index("c")
        pltpu.sync_copy(zero_hbm, zero_v)

        def data_body(i, _):
            base = (sub + i * _NSUB) * _TI
            pltpu.sync_copy(src_hbm.at[pl.ds(base, _TI)], sidx_v)
            pltpu.sync_copy(dst_hbm.at[pl.ds(base, _TI)], didx_v)
            gets = [
                pltpu.async_copy(x_hbm.at[sidx_v.at[j]],
                                 data_v.at[pl.ds(j * 128, 128)], sem_g)
                for j in range(_TI)
            ]
            for cp in gets:
                cp.wait()
            puts = [
                pltpu.async_copy(data_v.at[pl.ds(j * 128, 128)],
                                 out_hbm.at[didx_v.at[j]], sem_s)
                for j in range(_TI)
            ]
            for cp in puts:
                cp.wait()
            return 0

        my_data = (n_data_chunks - sub + _NSUB - 1) // _NSUB
        lax.fori_loop(0, my_data, data_body, 0)

        def pad_body(i, _):
            base = (sub + i * _NSUB) * _TI
            pltpu.sync_copy(pad_hbm.at[pl.ds(base, _TI)], sidx_v)
            puts = [
                pltpu.async_copy(zero_v, out_hbm.at[sidx_v.at[j]], sem_s)
                for j in range(_TI)
            ]
            for cp in puts:
                cp.wait()
            return 0

        my_pad = (n_pad_chunks - sub + _NSUB - 1) // _NSUB
        lax.fori_loop(0, my_pad, pad_body, 0)

    return k(x_flat, src2, dst2, pad2, zeros_rows)


def kernel(x, window_ids):
    b, n, c = x.shape
    wid = window_ids.reshape(-1)
    if wid.shape[0] > n:
        wid = wid[:n]
    elif wid.shape[0] < n:
        wid = jnp.concatenate(
            [wid, jnp.zeros(n - wid.shape[0], dtype=wid.dtype)])
    wid32 = wid.astype(jnp.int32)

    indices, counts, src, dst, pad = _build_lists(wid32, n, b)
    src2, n_data_chunks = _pad_chunk(src)
    dst2, _ = _pad_chunk(dst)
    pad2, n_pad_chunks = _pad_chunk(pad)

    x_flat = x.reshape(b * n, c)
    zeros_rows = jnp.zeros((128, 128), dtype=jnp.float32)
    out_rows = b * _W * _MW
    out = _sc_route(x_flat, src2, dst2, pad2, zeros_rows,
                    n_data_chunks, n_pad_chunks, out_rows)
    windows = out.reshape(b * _W, _MW, c)
    return windows, indices.astype(counts.dtype), counts


# speedup vs baseline: 22.8341x; 22.8341x over previous
import functools

import jax
import jax.numpy as jnp
from jax import lax
from jax.experimental import pallas as pl
from jax.experimental.pallas import tpu as pltpu
from jax.experimental.pallas import tpu_sc as plsc

_W = 162
_MW = 1088
_NC = 2
_NS = 16
_TI = 4
_T = _TI * 128
_ZROWS = 256


def _build_lists(wid, n, b):
    iota = jnp.arange(n, dtype=jnp.int32)
    sorted_wid, indices = lax.sort_key_val(wid, iota)
    wr = jnp.arange(_W, dtype=jnp.int32)
    starts = jnp.searchsorted(sorted_wid, wr, side="left").astype(jnp.int32)
    ends = jnp.searchsorted(sorted_wid, wr, side="right").astype(jnp.int32)
    counts = ends - starts
    new_seg = jnp.concatenate(
        [jnp.ones((1,), jnp.bool_), sorted_wid[1:] != sorted_wid[:-1]])
    seg_start = lax.cummax(jnp.where(new_seg, iota, 0))
    dst_sorted = sorted_wid * _MW + (iota - seg_start)

    boff = (jnp.arange(b, dtype=jnp.int32) * (_W * _MW))[:, None]
    src = jnp.arange(b, dtype=jnp.int32)[:, None] * n + indices[None, :]
    dst = boff + dst_sorted[None, :]
    return indices, counts, src, dst


def _pad_half(a, hp):
    h = a.shape[1]
    tail = jnp.broadcast_to(a[:, h - 1:h], (a.shape[0], hp - h))
    return jnp.concatenate([a, tail], axis=1).reshape(-1, 128)


def _sc_route(x_flat, src2, dst2, zeros_rows, n_half_chunks, rows_per_tile,
              out_rows):
    mesh = plsc.VectorSubcoreMesh(core_axis_name="c", subcore_axis_name="s")
    chunk_rows_per_half = n_half_chunks * _TI
    nz_full, nz_rem = rows_per_tile // _ZROWS, rows_per_tile % _ZROWS

    @functools.partial(
        pl.kernel,
        mesh=mesh,
        out_type=jax.ShapeDtypeStruct((out_rows, 128), jnp.float32),
        scratch_types=[
            pltpu.VMEM((_TI, 128), jnp.int32),
            pltpu.VMEM((_TI, 128), jnp.int32),
            pltpu.VMEM((_T, 128), jnp.float32),
            pltpu.VMEM((_ZROWS, 128), jnp.float32),
            pltpu.SemaphoreType.DMA,
            pltpu.SemaphoreType.DMA,
            pltpu.SemaphoreType.DMA,
        ],
    )
    def k(x_hbm, src_hbm, dst_hbm, zero_hbm, out_hbm,
          sidx_v, didx_v, data_v, zero_v, sem_g, sem_s, sem_z):
        core = lax.axis_index("c")
        tile = lax.axis_index("s")
        pltpu.sync_copy(zero_hbm, zero_v)

        zbase = (core * _NS + tile) * rows_per_tile

        def zero_body(i, _):
            cps = [
                pltpu.async_copy(
                    zero_v,
                    out_hbm.at[pl.ds(zbase + (i * 8 + u) * _ZROWS, _ZROWS)],
                    sem_z)
                for u in range(8)
            ]
            for cp in cps:
                cp.wait()
            return 0

        lax.fori_loop(0, nz_full // 8, zero_body, 0)
        tail = [
            pltpu.async_copy(
                zero_v,
                out_hbm.at[pl.ds(zbase + (nz_full // 8 * 8 + u) * _ZROWS,
                                 _ZROWS)], sem_z)
            for u in range(nz_full % 8)
        ]
        if nz_rem:
            tail.append(
                pltpu.async_copy(
                    zero_v.at[pl.ds(0, nz_rem)],
                    out_hbm.at[pl.ds(zbase + nz_full * _ZROWS, nz_rem)],
                    sem_z))
        for cp in tail:
            cp.wait()

        plsc.subcore_barrier()

        def data_body(i, _):
            j = tile + i * _NS
            base = core * chunk_rows_per_half + j * _TI
            pltpu.sync_copy(src_hbm.at[pl.ds(base, _TI)], sidx_v)
            pltpu.sync_copy(dst_hbm.at[pl.ds(base, _TI)], didx_v)
            gets = [
                pltpu.async_copy(x_hbm.at[sidx_v.at[u]],
                                 data_v.at[pl.ds(u * 128, 128)], sem_g)
                for u in range(_TI)
            ]
            for cp in gets:
                cp.wait()
            puts = [
                pltpu.async_copy(data_v.at[pl.ds(u * 128, 128)],
                                 out_hbm.at[didx_v.at[u]], sem_s)
                for u in range(_TI)
            ]
            for cp in puts:
                cp.wait()
            return 0

        my_chunks = (n_half_chunks - tile + _NS - 1) // _NS
        lax.fori_loop(0, my_chunks, data_body, 0)

    return k(x_flat, src2, dst2, zeros_rows)


def kernel(x, window_ids):
    b, n, c = x.shape
    wid = window_ids.reshape(-1)
    if wid.shape[0] > n:
        wid = wid[:n]
    elif wid.shape[0] < n:
        wid = jnp.concatenate(
            [wid, jnp.zeros(n - wid.shape[0], dtype=wid.dtype)])
    wid32 = wid.astype(jnp.int32)

    indices, counts, src, dst = _build_lists(wid32, n, b)

    h = (b // _NC) * n
    n_half_chunks = -(-h // _T)
    hp = n_half_chunks * _T
    src2 = _pad_half(src.reshape(_NC, h), hp)
    dst2 = _pad_half(dst.reshape(_NC, h), hp)

    out_rows = b * _W * _MW
    rows_per_tile = out_rows // (_NC * _NS)
    x_flat = x.reshape(b * n, c)
    zeros_rows = jnp.zeros((_ZROWS, 128), dtype=jnp.float32)
    out = _sc_route(x_flat, src2, dst2, zeros_rows, n_half_chunks,
                    rows_per_tile, out_rows)
    windows = out.reshape(b * _W, _MW, c)
    return windows, indices, counts
